# Initial kernel scaffold; baseline (speedup 1.0000x reference)
import jax
import jax.numpy as jnp
from jax.experimental import pallas as pl


def kernel(s, v, p, edge_index_local, d_local, a_local, r_local, e_local, edge_index_global, d_global, a_global, r_global, e_global, batch, ln_gamma, ln_beta, W1, b1, W2, b2, Wu1, bu1, Wu2, bu2, Wv):
    raise NotImplementedError("write your pallas kernel here")


# speedup vs baseline: 7.0145x; 7.0145x over previous
import functools

import jax
import jax.numpy as jnp
from jax import lax
from jax.experimental import pallas as pl
from jax.experimental.pallas import tpu as pltpu

_N = 50000
_E = 800000
_SDIM = 64
_VDIM = 16
_EDIM = 16
_L = 5
_CUTOFF = 5.0
_HID = 64
_VF = 3 * _VDIM

_BN = 2000
_BE = 2000

_PREC = lax.Precision.HIGHEST


def _stats_body(s_ref, v_ref, o_ref):
    i = pl.program_id(0)
    s = s_ref[...]
    sc = s - jnp.mean(s, axis=1, keepdims=True)
    ps = jnp.sum(sc * sc) * (1.0 / _SDIM)
    vb = v_ref[...]
    pv = jnp.sum(vb * vb) * (1.0 / _VF)

    @pl.when(i == 0)
    def _():
        o_ref[...] = jnp.zeros_like(o_ref)

    col = lax.broadcasted_iota(jnp.int32, (1, 128), 1)
    o_ref[...] += jnp.where(col == 0, ps, 0.0) + jnp.where(col == 1, pv, 0.0)


def _node_stats(s, vf):
    return pl.pallas_call(
        _stats_body,
        grid=(_N // _BN,),
        in_specs=[
            pl.BlockSpec((_BN, _SDIM), lambda i: (i, 0)),
            pl.BlockSpec((_BN, _VF), lambda i: (i, 0)),
        ],
        out_specs=pl.BlockSpec((1, 128), lambda i: (0, 0)),
        out_shape=jax.ShapeDtypeStruct((1, 128), jnp.float32),
    )(s, vf)


def _apply_body(s_ref, v_ref, st_ref, g_ref, b_ref, sn_ref, vn_ref):
    st = st_ref[...]
    inv_s = lax.rsqrt(st[0, 0] * (1.0 / _N) + 1e-6)
    inv_v = lax.rsqrt(st[0, 1] * (1.0 / _N) + 1e-6)
    s = s_ref[...]
    sc = s - jnp.mean(s, axis=1, keepdims=True)
    sn_ref[...] = sc * inv_s * g_ref[...] + b_ref[...]
    vn_ref[...] = v_ref[...] * inv_v


def _node_apply(s, vf, stats, gamma, beta):
    return pl.pallas_call(
        _apply_body,
        grid=(_N // _BN,),
        in_specs=[
            pl.BlockSpec((_BN, _SDIM), lambda i: (i, 0)),
            pl.BlockSpec((_BN, _VF), lambda i: (i, 0)),
            pl.BlockSpec((1, 128), lambda i: (0, 0)),
            pl.BlockSpec((1, _SDIM), lambda i: (0, 0)),
            pl.BlockSpec((1, _SDIM), lambda i: (0, 0)),
        ],
        out_specs=[
            pl.BlockSpec((_BN, _SDIM), lambda i: (i, 0)),
            pl.BlockSpec((_BN, _VF), lambda i: (i, 0)),
        ],
        out_shape=[
            jax.ShapeDtypeStruct((_N, _SDIM), jnp.float32),
            jax.ShapeDtypeStruct((_N, _VF), jnp.float32),
        ],
    )(s, vf, stats, gamma, beta)


def _silu(x):
    return x * jax.nn.sigmoid(x)


def _mlp_body_v(ssrc_ref, sdst_ref, ae_ref, aux_ref, vsrc_ref,
                w1a_ref, w1b_ref, w1ce_ref, w1d_ref, b1_ref,
                w2s_ref, b2s_ref, w2g1_ref, b2g1_ref, w2g2_ref, b2g2_ref,
                ms_ref, mv_ref):
    _mlp_common(ssrc_ref, sdst_ref, ae_ref, aux_ref, vsrc_ref,
                w1a_ref, w1b_ref, w1ce_ref, w1d_ref, b1_ref,
                w2s_ref, b2s_ref, w2g1_ref, b2g1_ref, w2g2_ref, b2g2_ref,
                ms_ref, mv_ref)


def _mlp_body_nov(ssrc_ref, sdst_ref, ae_ref, aux_ref,
                  w1a_ref, w1b_ref, w1ce_ref, w1d_ref, b1_ref,
                  w2s_ref, b2s_ref, w2g1_ref, b2g1_ref, w2g2_ref, b2g2_ref,
                  ms_ref, mv_ref):
    _mlp_common(ssrc_ref, sdst_ref, ae_ref, aux_ref, None,
                w1a_ref, w1b_ref, w1ce_ref, w1d_ref, b1_ref,
                w2s_ref, b2s_ref, w2g1_ref, b2g1_ref, w2g2_ref, b2g2_ref,
                ms_ref, mv_ref)


def _mlp_common(ssrc_ref, sdst_ref, ae_ref, aux_ref, vsrc_ref,
                w1a_ref, w1b_ref, w1ce_ref, w1d_ref, b1_ref,
                w2s_ref, b2s_ref, w2g1_ref, b2g1_ref, w2g2_ref, b2g2_ref,
                ms_ref, mv_ref):
    dcol = aux_ref[:, 3:4]
    dd = jnp.clip(dcol, 0.0, _CUTOFF)
    C = 0.5 * (jnp.cos(jnp.pi * dd / _CUTOFF) + 1.0)
    C = jnp.where(dcol < _CUTOFF, C, 0.0)

    h1 = (jnp.dot(ssrc_ref[...], w1a_ref[...], precision=_PREC)
          + jnp.dot(sdst_ref[...], w1b_ref[...], precision=_PREC)
          + jnp.dot(ae_ref[...], w1ce_ref[...], precision=_PREC)
          + dcol * w1d_ref[...] + b1_ref[...])
    h = _silu(h1)
    ms_ref[...] = (jnp.dot(h, w2s_ref[...], precision=_PREC) + b2s_ref[...]) * C
    g1 = jnp.dot(h, w2g1_ref[...], precision=_PREC) + b2g1_ref[...]
    g2 = jnp.dot(h, w2g2_ref[...], precision=_PREC) + b2g2_ref[...]
    for j in range(3):
        rj = aux_ref[:, j:j + 1]
        mvj = g2 * rj
        if vsrc_ref is not None:
            mvj = mvj + g1 * vsrc_ref[:, j * _VDIM:(j + 1) * _VDIM]
        mv_ref[:, j * _VDIM:(j + 1) * _VDIM] = mvj * C


def _edge_mlp(ssrc, sdst, ae, auxT, vsrc, wts):
    (w1a, w1b, w1ce, w1d, b1, w2s, b2s, w2g1, b2g1, w2g2, b2g2) = wts
    full = lambda a: pl.BlockSpec(a.shape, lambda i: tuple(0 for _ in a.shape))
    in_specs = [
        pl.BlockSpec((_BE, _SDIM), lambda i: (i, 0)),
        pl.BlockSpec((_BE, _SDIM), lambda i: (i, 0)),
        pl.BlockSpec((_BE, 2 * _EDIM), lambda i: (i, 0)),
        pl.BlockSpec((_BE, 8), lambda i: (i, 0)),
    ]
    args = [ssrc, sdst, ae, auxT]
    if vsrc is not None:
        in_specs.append(pl.BlockSpec((_BE, _VF), lambda i: (i, 0)))
        args.append(vsrc)
        body = _mlp_body_v
    else:
        body = _mlp_body_nov
    for w in (w1a, w1b, w1ce, w1d, b1, w2s, b2s, w2g1, b2g1, w2g2, b2g2):
        in_specs.append(full(w))
        args.append(w)
    return pl.pallas_call(
        body,
        grid=(_E // _BE,),
        in_specs=in_specs,
        out_specs=[
            pl.BlockSpec((_BE, _SDIM), lambda i: (i, 0)),
            pl.BlockSpec((_BE, _VF), lambda i: (i, 0)),
        ],
        out_shape=[
            jax.ShapeDtypeStruct((_E, _SDIM), jnp.float32),
            jax.ShapeDtypeStruct((_E, _VF), jnp.float32),
        ],
    )(*args)


def _update_body_mlp(sn_ref, sagg_ref, vn_ref, vagg_ref, cnt_ref,
                     wu1a_ref, wu1b_ref, bu1_ref, wu2_ref, bu2_ref, wv_ref,
                     so_ref, vo_ref):
    cinv = 1.0 / jnp.maximum(cnt_ref[...], 1.0)
    vn = vn_ref[...]
    va = vagg_ref[...]
    for j in range(3):
        sl = slice(j * _VDIM, (j + 1) * _VDIM)
        vo_ref[:, sl] = vn[:, sl] + jnp.dot(va[:, sl] * cinv, wv_ref[...],
                                            precision=_PREC)
    sn = sn_ref[...]
    u = _silu(jnp.dot(sn, wu1a_ref[...], precision=_PREC)
              + jnp.dot(sagg_ref[...], wu1b_ref[...], precision=_PREC)
              + bu1_ref[...])
    so_ref[...] = sn + jnp.dot(u, wu2_ref[...], precision=_PREC) + bu2_ref[...]


def _update_body_last(sn_ref, sagg_ref, vn_ref, vagg_ref, cnt_ref, wv_ref,
                      so_ref, vo_ref):
    cinv = 1.0 / jnp.maximum(cnt_ref[...], 1.0)
    vn = vn_ref[...]
    va = vagg_ref[...]
    for j in range(3):
        sl = slice(j * _VDIM, (j + 1) * _VDIM)
        vo_ref[:, sl] = vn[:, sl] + jnp.dot(va[:, sl] * cinv, wv_ref[...],
                                            precision=_PREC)
    so_ref[...] = sn_ref[...] + sagg_ref[...]


def _node_update(sn, sagg, vn, vagg, cnt16, wu, wv, last):
    full = lambda a: pl.BlockSpec(a.shape, lambda i: tuple(0 for _ in a.shape))
    in_specs = [
        pl.BlockSpec((_BN, _SDIM), lambda i: (i, 0)),
        pl.BlockSpec((_BN, _SDIM), lambda i: (i, 0)),
        pl.BlockSpec((_BN, _VF), lambda i: (i, 0)),
        pl.BlockSpec((_BN, _VF), lambda i: (i, 0)),
        pl.BlockSpec((_BN, _VDIM), lambda i: (i, 0)),
    ]
    args = [sn, sagg, vn, vagg, cnt16]
    if last:
        body = _update_body_last
    else:
        body = _update_body_mlp
        for w in wu:
            in_specs.append(full(w))
            args.append(w)
    in_specs.append(full(wv))
    args.append(wv)
    return pl.pallas_call(
        body,
        grid=(_N // _BN,),
        in_specs=in_specs,
        out_specs=[
            pl.BlockSpec((_BN, _SDIM), lambda i: (i, 0)),
            pl.BlockSpec((_BN, _VF), lambda i: (i, 0)),
        ],
        out_shape=[
            jax.ShapeDtypeStruct((_N, _SDIM), jnp.float32),
            jax.ShapeDtypeStruct((_N, _VF), jnp.float32),
        ],
    )(*args)


def _gather2(tab_a, tab_b, idx_a, idx_b):
    return jnp.take(tab_a, idx_a, axis=0), jnp.take(tab_b, idx_b, axis=0)


def _gather3(tab_a, tab_b, tab_c, idx_a, idx_b):
    return (jnp.take(tab_a, idx_a, axis=0), jnp.take(tab_b, idx_b, axis=0),
            jnp.take(tab_c, idx_a, axis=0))


def _scatter_add(data, idx, d):
    return jax.ops.segment_sum(data, idx, num_segments=_N)


def kernel(s, v, p, edge_index_local, d_local, a_local, r_local, e_local,
           edge_index_global, d_global, a_global, r_global, e_global, batch,
           ln_gamma, ln_beta, W1, b1, W2, b2, Wu1, bu1, Wu2, bu2, Wv):
    src = edge_index_local[0]
    dst = edge_index_local[1]
    vf = v.reshape(_N, _VF)

    ae = jnp.concatenate([a_local, e_local], axis=1)
    auxT = jnp.concatenate(
        [r_local, d_local[:, None], jnp.zeros((_E, 4), jnp.float32)], axis=1)

    cnt16 = _scatter_add(jnp.ones((_E, _VDIM), jnp.float32), dst, _VDIM)

    for i in range(_L):
        stats = _node_stats(s, vf)
        sn, vn = _node_apply(s, vf, stats, ln_gamma[i:i + 1], ln_beta[i:i + 1])

        w1 = W1[i]
        wts = (w1[:_SDIM], w1[_SDIM:2 * _SDIM], w1[2 * _SDIM:2 * _SDIM + 32],
               w1[2 * _SDIM + 32:2 * _SDIM + 33], b1[i:i + 1],
               W2[i][:, :_SDIM], b2[i:i + 1, :_SDIM],
               W2[i][:, _SDIM:_SDIM + _VDIM], b2[i:i + 1, _SDIM:_SDIM + _VDIM],
               W2[i][:, _SDIM + _VDIM:], b2[i:i + 1, _SDIM + _VDIM:])

        if i > 0:
            ssrc, sdst, vsrc = _gather3(sn, sn, vn, src, dst)
        else:
            ssrc, sdst = _gather2(sn, sn, src, dst)
            vsrc = None

        ms, mv = _edge_mlp(ssrc, sdst, ae, auxT, vsrc, wts)

        sagg = _scatter_add(ms, dst, _SDIM)
        vagg = _scatter_add(mv, dst, _VF)

        wu = (Wu1[i][:_SDIM], Wu1[i][_SDIM:], bu1[i:i + 1], Wu2[i],
              bu2[i:i + 1])
        s, vf = _node_update(sn, sagg, vn, vagg, cnt16, wu, Wv[i],
                             last=(i == _L - 1))

    return (s, vf.reshape(_N, 3, _VDIM))


# speedup vs baseline: 14.9798x; 2.1355x over previous
import functools

import jax
import jax.numpy as jnp
from jax import lax
from jax.experimental import pallas as pl
from jax.experimental.pallas import tpu as pltpu
from jax.experimental.pallas import tpu_sc as plsc

_N = 50000
_E = 800000
_SDIM = 64
_VDIM = 16
_EDIM = 16
_L = 5
_CUTOFF = 5.0
_HID = 64
_VF = 3 * _VDIM

_BN = 2000
_BE = 2000

_PREC = lax.Precision.HIGHEST


def _stats_body(s_ref, v_ref, o_ref):
    i = pl.program_id(0)
    s = s_ref[...]
    sc = s - jnp.mean(s, axis=1, keepdims=True)
    ps = jnp.sum(sc * sc) * (1.0 / _SDIM)
    vb = v_ref[...]
    pv = jnp.sum(vb * vb) * (1.0 / _VF)

    @pl.when(i == 0)
    def _():
        o_ref[...] = jnp.zeros_like(o_ref)

    col = lax.broadcasted_iota(jnp.int32, (1, 128), 1)
    o_ref[...] += jnp.where(col == 0, ps, 0.0) + jnp.where(col == 1, pv, 0.0)


def _node_stats(s, vf):
    return pl.pallas_call(
        _stats_body,
        grid=(_N // _BN,),
        in_specs=[
            pl.BlockSpec((_BN, _SDIM), lambda i: (i, 0)),
            pl.BlockSpec((_BN, _VF), lambda i: (i, 0)),
        ],
        out_specs=pl.BlockSpec((1, 128), lambda i: (0, 0)),
        out_shape=jax.ShapeDtypeStruct((1, 128), jnp.float32),
    )(s, vf)


def _apply_body(s_ref, v_ref, st_ref, g_ref, b_ref, tbl_ref):
    st = st_ref[...]
    inv_s = lax.rsqrt(st[0, 0] * (1.0 / _N) + 1e-6)
    inv_v = lax.rsqrt(st[0, 1] * (1.0 / _N) + 1e-6)
    s = s_ref[...]
    sc = s - jnp.mean(s, axis=1, keepdims=True)
    tbl_ref[:, :_SDIM] = sc * inv_s * g_ref[...] + b_ref[...]
    tbl_ref[:, _SDIM:_SDIM + _VF] = v_ref[...] * inv_v
    tbl_ref[:, _SDIM + _VF:] = jnp.zeros((s.shape[0], 128 - _SDIM - _VF),
                                         jnp.float32)


def _node_apply(s, vf, stats, gamma, beta):
    return pl.pallas_call(
        _apply_body,
        grid=(_N // _BN,),
        in_specs=[
            pl.BlockSpec((_BN, _SDIM), lambda i: (i, 0)),
            pl.BlockSpec((_BN, _VF), lambda i: (i, 0)),
            pl.BlockSpec((1, 128), lambda i: (0, 0)),
            pl.BlockSpec((1, _SDIM), lambda i: (0, 0)),
            pl.BlockSpec((1, _SDIM), lambda i: (0, 0)),
        ],
        out_specs=pl.BlockSpec((_BN, 128), lambda i: (i, 0)),
        out_shape=jax.ShapeDtypeStruct((_N, 128), jnp.float32),
    )(s, vf, stats, gamma, beta)


def _silu(x):
    return x * jax.nn.sigmoid(x)


def _mlp_body(has_v, pad_val, gsrc_ref, sdst_ref, ae_ref, aux_ref,
              w1a_ref, w1b_ref, w1ce_ref, w1d_ref, b1_ref,
              w2s_ref, b2s_ref, w2g1_ref, b2g1_ref, w2g2_ref, b2g2_ref,
              ms_ref):
    dcol = aux_ref[:, 3:4]
    dd = jnp.clip(dcol, 0.0, _CUTOFF)
    C = 0.5 * (jnp.cos(jnp.pi * dd / _CUTOFF) + 1.0)
    C = jnp.where(dcol < _CUTOFF, C, 0.0)

    ssrc = gsrc_ref[:, :_SDIM]
    h1 = (jnp.dot(ssrc, w1a_ref[...], precision=_PREC)
          + jnp.dot(sdst_ref[:, :_SDIM], w1b_ref[...], precision=_PREC)
          + jnp.dot(ae_ref[...], w1ce_ref[...], precision=_PREC)
          + dcol * w1d_ref[...] + b1_ref[...])
    h = _silu(h1)
    ms_ref[:, :_SDIM] = (jnp.dot(h, w2s_ref[...], precision=_PREC)
                         + b2s_ref[...]) * C
    g1 = jnp.dot(h, w2g1_ref[...], precision=_PREC) + b2g1_ref[...]
    g2 = jnp.dot(h, w2g2_ref[...], precision=_PREC) + b2g2_ref[...]
    for j in range(3):
        rj = aux_ref[:, j:j + 1]
        mvj = g2 * rj
        if has_v:
            mvj = mvj + g1 * gsrc_ref[:, _SDIM + j * _VDIM:
                                      _SDIM + (j + 1) * _VDIM]
        ms_ref[:, _SDIM + j * _VDIM:_SDIM + (j + 1) * _VDIM] = mvj * C
    ms_ref[:, _SDIM + 3 * _VDIM:] = jnp.full((h.shape[0], _VDIM), pad_val,
                                             jnp.float32)


def _edge_mlp(gsrc, sdst, ae, auxT, has_v, wts, pad_val):
    (w1a, w1b, w1ce, w1d, b1, w2s, b2s, w2g1, b2g1, w2g2, b2g2) = wts
    full = lambda a: pl.BlockSpec(a.shape, lambda i: tuple(0 for _ in a.shape))
    in_specs = [
        pl.BlockSpec((_BE, 128), lambda i: (i, 0)),
        pl.BlockSpec((_BE, 128), lambda i: (i, 0)),
        pl.BlockSpec((_BE, 2 * _EDIM), lambda i: (i, 0)),
        pl.BlockSpec((_BE, 8), lambda i: (i, 0)),
    ]
    args = [gsrc, sdst, ae, auxT]
    body = functools.partial(_mlp_body, has_v, pad_val)
    for w in (w1a, w1b, w1ce, w1d, b1, w2s, b2s, w2g1, b2g1, w2g2, b2g2):
        in_specs.append(full(w))
        args.append(w)
    return pl.pallas_call(
        body,
        grid=(_E // _BE,),
        in_specs=in_specs,
        out_specs=pl.BlockSpec((_BE, 128), lambda i: (i, 0)),
        out_shape=jax.ShapeDtypeStruct((_E, 128), jnp.float32),
    )(*args)


def _update_body_mlp(tbl_ref, sagg_ref, vagg_ref, cnt_ref,
                     wu1a_ref, wu1b_ref, bu1_ref, wu2_ref, bu2_ref, wv_ref,
                     so_ref, vo_ref):
    cinv = 1.0 / jnp.maximum(cnt_ref[...], 1.0)
    va = vagg_ref[...]
    for j in range(3):
        sl = slice(j * _VDIM, (j + 1) * _VDIM)
        vo_ref[:, sl] = (tbl_ref[:, _SDIM + j * _VDIM:_SDIM + (j + 1) * _VDIM]
                         + jnp.dot(va[:, sl] * cinv, wv_ref[...],
                                   precision=_PREC))
    sn = tbl_ref[:, :_SDIM]
    u = _silu(jnp.dot(sn, wu1a_ref[...], precision=_PREC)
              + jnp.dot(sagg_ref[:, :_SDIM], wu1b_ref[...], precision=_PREC)
              + bu1_ref[...])
    so_ref[...] = sn + jnp.dot(u, wu2_ref[...], precision=_PREC) + bu2_ref[...]


def _update_body_last(tbl_ref, sagg_ref, vagg_ref, cnt_ref, wv_ref,
                      so_ref, vo_ref):
    cinv = 1.0 / jnp.maximum(cnt_ref[...], 1.0)
    va = vagg_ref[...]
    for j in range(3):
        sl = slice(j * _VDIM, (j + 1) * _VDIM)
        vo_ref[:, sl] = (tbl_ref[:, _SDIM + j * _VDIM:_SDIM + (j + 1) * _VDIM]
                         + jnp.dot(va[:, sl] * cinv, wv_ref[...],
                                   precision=_PREC))
    so_ref[...] = tbl_ref[:, :_SDIM] + sagg_ref[:, :_SDIM]


def _node_update(tbl, sagg, vagg, cnt16, wu, wv, last):
    full = lambda a: pl.BlockSpec(a.shape, lambda i: tuple(0 for _ in a.shape))
    in_specs = [
        pl.BlockSpec((_BN, 128), lambda i: (i, 0)),
        pl.BlockSpec((_BN, 128), lambda i: (i, 0)),
        pl.BlockSpec((_BN, 128), lambda i: (i, 0)),
        pl.BlockSpec((_BN, _VDIM), lambda i: (i, 0)),
    ]
    args = [tbl, sagg, vagg, cnt16]
    if last:
        body = _update_body_last
    else:
        body = _update_body_mlp
        for w in wu:
            in_specs.append(full(w))
            args.append(w)
    in_specs.append(full(wv))
    args.append(wv)
    return pl.pallas_call(
        body,
        grid=(_N // _BN,),
        in_specs=in_specs,
        out_specs=[
            pl.BlockSpec((_BN, _SDIM), lambda i: (i, 0)),
            pl.BlockSpec((_BN, _VF), lambda i: (i, 0)),
        ],
        out_shape=[
            jax.ShapeDtypeStruct((_N, _SDIM), jnp.float32),
            jax.ShapeDtypeStruct((_N, _VF), jnp.float32),
        ],
    )(*args)


_WS = 200
_SNW = _E // 16 // _WS
_NT = 3128
_NT_LAST = _N - 15 * _NT
_WG = 200
_GNW = _E // 32 // _WG

_SC_MESH = dict(core_axis_name="c", subcore_axis_name="s")


def _scatter64_body(cb, data_hbm, idx_hbm, z_hbm, out_hbm, acc,
                    ix0, ix1, upd0, upd1, sf0, sf1, ss0, ss1):
    co = lax.axis_index("c")
    sid = lax.axis_index("s")
    colbase = co * 32
    ebase = sid * (_E // 16)

    @pl.when(sid < 15)
    def _():
        pltpu.sync_copy(z_hbm, acc.at[pl.ds(sid * _NT, _NT)])

    @pl.when(sid == 15)
    def _():
        pltpu.sync_copy(z_hbm.at[pl.ds(0, _NT_LAST)],
                        acc.at[pl.ds(15 * _NT, _NT_LAST)])

    plsc.subcore_barrier()

    def fetch(k, ix, upd, sem):
        pltpu.async_copy(idx_hbm.at[pl.ds(ebase + k * _WS, _WS)], ix, sem)
        pltpu.async_copy(
            data_hbm.at[pl.ds(ebase + k * _WS, _WS),
                        pl.ds(cb + colbase, 32)], upd, sem)

    def wait_f(ix, upd, sem):
        pltpu.make_async_copy(idx_hbm.at[pl.ds(0, _WS)], ix, sem).wait()
        pltpu.make_async_copy(
            data_hbm.at[pl.ds(0, _WS), pl.ds(cb + colbase, 32)],
            upd, sem).wait()

    def add(ix, upd, sem):
        pltpu.async_copy(upd, acc.at[ix], sem, add=True)

    def wait_s(ix, upd, sem):
        pltpu.make_async_copy(upd, acc.at[ix], sem).wait()

    fetch(0, ix0, upd0, sf0)
    fetch(1, ix1, upd1, sf1)

    def step(k2, carry):
        a = 2 * k2
        wait_f(ix0, upd0, sf0)
        add(ix0, upd0, ss0)
        wait_f(ix1, upd1, sf1)
        add(ix1, upd1, ss1)
        wait_s(ix0, upd0, ss0)
        fetch(a + 2, ix0, upd0, sf0)
        wait_s(ix1, upd1, ss1)
        fetch(a + 3, ix1, upd1, sf1)
        return carry

    lax.fori_loop(0, _SNW // 2 - 1, step, 0)
    wait_f(ix0, upd0, sf0)
    add(ix0, upd0, ss0)
    wait_f(ix1, upd1, sf1)
    add(ix1, upd1, ss1)
    wait_s(ix0, upd0, ss0)
    wait_s(ix1, upd1, ss1)
    plsc.subcore_barrier()

    @pl.when(sid < 15)
    def _():
        pltpu.sync_copy(acc.at[pl.ds(sid * _NT, _NT)],
                        out_hbm.at[pl.ds(sid * _NT, _NT),
                                   pl.ds(colbase, 32)])

    @pl.when(sid == 15)
    def _():
        pltpu.sync_copy(acc.at[pl.ds(15 * _NT, _NT_LAST)],
                        out_hbm.at[pl.ds(15 * _NT, _NT_LAST),
                                   pl.ds(colbase, 32)])


def _sc_scatter64(data, dst, zeros_t, cb):
    return pl.kernel(
        functools.partial(_scatter64_body, cb),
        out_type=jax.ShapeDtypeStruct((_N, 128), jnp.float32),
        mesh=plsc.VectorSubcoreMesh(**_SC_MESH),
        scratch_types=[
            pltpu.VMEM_SHARED((_N, 32), jnp.float32),
            pltpu.VMEM((_WS,), jnp.int32),
            pltpu.VMEM((_WS,), jnp.int32),
            pltpu.VMEM((_WS, 32), jnp.float32),
            pltpu.VMEM((_WS, 32), jnp.float32),
            pltpu.SemaphoreType.DMA,
            pltpu.SemaphoreType.DMA,
            pltpu.SemaphoreType.DMA,
            pltpu.SemaphoreType.DMA,
        ],
        compiler_params=pltpu.CompilerParams(use_tc_tiling_on_sc=False),
    )(data, dst, zeros_t)


_EPW = _E // 32


def _gather_body(tbl_hbm, idx_hbm, out_hbm, ixf, b0, b1,
                 sg0, sg1, sw0, sw1):
    co = lax.axis_index("c")
    sid = lax.axis_index("s")
    wid = sid * 2 + co
    ebase = wid * _EPW

    pltpu.sync_copy(idx_hbm.at[pl.ds(ebase, _EPW)], ixf)

    def g_fire(k, b, sem):
        pltpu.async_copy(tbl_hbm.at[ixf.at[pl.ds(k * _WG, _WG)]], b, sem)

    def g_wait(b, sem):
        pltpu.make_async_copy(
            tbl_hbm.at[ixf.at[pl.ds(0, _WG)]], b, sem).wait()

    def w_fire(k, b, sem):
        pltpu.async_copy(b, out_hbm.at[pl.ds(ebase + k * _WG, _WG)], sem)

    def w_wait(b, sem):
        pltpu.make_async_copy(b, out_hbm.at[pl.ds(0, _WG)], sem).wait()

    g_fire(0, b0, sg0)
    g_wait(b0, sg0)
    w_fire(0, b0, sw0)
    g_fire(1, b1, sg1)
    w_wait(b0, sw0)
    g_fire(2, b0, sg0)

    def step(k2, carry):
        a = 1 + 2 * k2
        g_wait(b1, sg1)
        w_fire(a, b1, sw1)
        g_wait(b0, sg0)
        w_fire(a + 1, b0, sw0)
        w_wait(b1, sw1)
        g_fire(a + 2, b1, sg1)
        w_wait(b0, sw0)
        g_fire(a + 3, b0, sg0)
        return carry

    lax.fori_loop(0, (_GNW - 1) // 2 - 1, step, 0)
    a = _GNW - 2
    g_wait(b1, sg1)
    w_fire(a, b1, sw1)
    g_wait(b0, sg0)
    w_fire(a + 1, b0, sw0)
    w_wait(b1, sw1)
    w_wait(b0, sw0)


def _sc_gather1(tbl, idx):
    return pl.kernel(
        _gather_body,
        out_type=jax.ShapeDtypeStruct((_E, 128), jnp.float32),
        mesh=plsc.VectorSubcoreMesh(**_SC_MESH),
        scratch_types=[
            pltpu.VMEM((_EPW,), jnp.int32),
            pltpu.VMEM((_WG, 128), jnp.float32),
            pltpu.VMEM((_WG, 128), jnp.float32),
            pltpu.SemaphoreType.DMA,
            pltpu.SemaphoreType.DMA,
            pltpu.SemaphoreType.DMA,
            pltpu.SemaphoreType.DMA,
        ],
    )(tbl, idx)


def kernel(s, v, p, edge_index_local, d_local, a_local, r_local, e_local,
           edge_index_global, d_global, a_global, r_global, e_global, batch,
           ln_gamma, ln_beta, W1, b1, W2, b2, Wu1, bu1, Wu2, bu2, Wv):
    src = edge_index_local[0]
    dst = edge_index_local[1]
    vf = v.reshape(_N, _VF)

    ae = jnp.concatenate([a_local, e_local], axis=1)
    auxT = jnp.concatenate(
        [r_local, d_local[:, None], jnp.zeros((_E, 4), jnp.float32)], axis=1)

    zeros_t = jnp.zeros((_NT, 32), jnp.float32)

    cnt16 = None
    for i in range(_L):
        stats = _node_stats(s, vf)
        tbl = _node_apply(s, vf, stats, ln_gamma[i:i + 1], ln_beta[i:i + 1])

        w1 = W1[i]
        wts = (w1[:_SDIM], w1[_SDIM:2 * _SDIM], w1[2 * _SDIM:2 * _SDIM + 32],
               w1[2 * _SDIM + 32:2 * _SDIM + 33], b1[i:i + 1],
               W2[i][:, :_SDIM], b2[i:i + 1, :_SDIM],
               W2[i][:, _SDIM:_SDIM + _VDIM], b2[i:i + 1, _SDIM:_SDIM + _VDIM],
               W2[i][:, _SDIM + _VDIM:], b2[i:i + 1, _SDIM + _VDIM:])

        gsrc = _sc_gather1(tbl, src)
        sdst = _sc_gather1(tbl, dst)

        msmv = _edge_mlp(gsrc, sdst, ae, auxT, i > 0, wts,
                         pad_val=1.0 if i == 0 else 0.0)

        sagg = _sc_scatter64(msmv, dst, zeros_t, cb=0)
        vagg = _sc_scatter64(msmv, dst, zeros_t, cb=64)
        if i == 0:
            cnt16 = vagg[:, _VF:_SDIM]

        wu = (Wu1[i][:_SDIM], Wu1[i][_SDIM:], bu1[i:i + 1], Wu2[i],
              bu2[i:i + 1])
        s, vf = _node_update(tbl, sagg, vagg, cnt16, wu, Wv[i],
                             last=(i == _L - 1))

    return (s, vf.reshape(_N, 3, _VDIM))


# speedup vs baseline: 37.3394x; 2.4926x over previous
import functools

import jax
import jax.numpy as jnp
from jax import lax
from jax.experimental import pallas as pl
from jax.experimental.pallas import tpu as pltpu
from jax.experimental.pallas import tpu_sc as plsc

_N = 50000
_E = 800000
_SDIM = 64
_VDIM = 16
_EDIM = 16
_L = 5
_CUTOFF = 5.0
_HID = 64
_VF = 3 * _VDIM

_BN = 2000
_BE = 2000

_PREC = lax.Precision.DEFAULT


def _stats_body(s_ref, v_ref, o_ref):
    i = pl.program_id(0)
    s = s_ref[...]
    sc = s - jnp.mean(s, axis=1, keepdims=True)
    ps = jnp.sum(sc * sc) * (1.0 / _SDIM)
    vb = v_ref[...]
    pv = jnp.sum(vb * vb) * (1.0 / _VF)

    @pl.when(i == 0)
    def _():
        o_ref[...] = jnp.zeros_like(o_ref)

    col = lax.broadcasted_iota(jnp.int32, (1, 128), 1)
    o_ref[...] += jnp.where(col == 0, ps, 0.0) + jnp.where(col == 1, pv, 0.0)


def _node_stats(s, vf):
    return pl.pallas_call(
        _stats_body,
        grid=(_N // _BN,),
        in_specs=[
            pl.BlockSpec((_BN, _SDIM), lambda i: (i, 0)),
            pl.BlockSpec((_BN, _VF), lambda i: (i, 0)),
        ],
        out_specs=pl.BlockSpec((1, 128), lambda i: (0, 0)),
        out_shape=jax.ShapeDtypeStruct((1, 128), jnp.float32),
    )(s, vf)


def _apply_body(s_ref, v_ref, st_ref, g_ref, b_ref, tbl_ref):
    st = st_ref[...]
    inv_s = lax.rsqrt(st[0, 0] * (1.0 / _N) + 1e-6)
    inv_v = lax.rsqrt(st[0, 1] * (1.0 / _N) + 1e-6)
    s = s_ref[...]
    sc = s - jnp.mean(s, axis=1, keepdims=True)
    tbl_ref[:, :_SDIM] = sc * inv_s * g_ref[...] + b_ref[...]
    tbl_ref[:, _SDIM:_SDIM + _VF] = v_ref[...] * inv_v
    tbl_ref[:, _SDIM + _VF:] = jnp.zeros((s.shape[0], 128 - _SDIM - _VF),
                                         jnp.float32)


def _node_apply(s, vf, stats, gamma, beta):
    return pl.pallas_call(
        _apply_body,
        grid=(_N // _BN,),
        in_specs=[
            pl.BlockSpec((_BN, _SDIM), lambda i: (i, 0)),
            pl.BlockSpec((_BN, _VF), lambda i: (i, 0)),
            pl.BlockSpec((1, 128), lambda i: (0, 0)),
            pl.BlockSpec((1, _SDIM), lambda i: (0, 0)),
            pl.BlockSpec((1, _SDIM), lambda i: (0, 0)),
        ],
        out_specs=pl.BlockSpec((_BN, 128), lambda i: (i, 0)),
        out_shape=jax.ShapeDtypeStruct((_N, 128), jnp.float32),
    )(s, vf, stats, gamma, beta)


def _silu(x):
    return x * jax.nn.sigmoid(x)


def _prep_body(a_ref, e_ref, d_ref, r_ref, aed_ref, rc_ref):
    d = d_ref[...]
    dd = jnp.clip(d, 0.0, _CUTOFF)
    C = 0.5 * (jnp.cos(jnp.pi * dd / _CUTOFF) + 1.0)
    C = jnp.where(d < _CUTOFF, C, 0.0)
    aed_ref[:, 0:_EDIM] = a_ref[...]
    aed_ref[:, _EDIM:2 * _EDIM] = e_ref[...]
    aed_ref[:, 32:33] = d
    aed_ref[:, 33:34] = C
    aed_ref[:, 34:48] = jnp.zeros((d.shape[0], 14), jnp.float32)
    ones16 = jnp.ones((1, _VDIM), jnp.float32)
    for j in range(3):
        rc_ref[:, j * _VDIM:(j + 1) * _VDIM] = (C * r_ref[:, j:j + 1]) * ones16


def _edge_prep(a, e, d2, r):
    return pl.pallas_call(
        _prep_body,
        grid=(_E // _BE,),
        in_specs=[
            pl.BlockSpec((_BE, _EDIM), lambda i: (i, 0)),
            pl.BlockSpec((_BE, _EDIM), lambda i: (i, 0)),
            pl.BlockSpec((_BE, 1), lambda i: (i, 0)),
            pl.BlockSpec((_BE, 3), lambda i: (i, 0)),
        ],
        out_specs=[
            pl.BlockSpec((_BE, 48), lambda i: (i, 0)),
            pl.BlockSpec((_BE, 48), lambda i: (i, 0)),
        ],
        out_shape=[
            jax.ShapeDtypeStruct((_E, 48), jnp.float32),
            jax.ShapeDtypeStruct((_E, 48), jnp.float32),
        ],
    )(a, e, d2, r)


def _mlp_body(has_v, pad_val, gsrc_ref, sdst_ref, aed_ref, rc_ref,
              w1a_ref, w1b_ref, w1ce_ref, b1_ref,
              w2s_ref, b2s_ref, w2g1_ref, b2g1_ref, w2g2_ref, b2g2_ref,
              ms_ref):
    C = aed_ref[:, 33:34]
    h1 = (jnp.dot(gsrc_ref[:, :_SDIM], w1a_ref[...], precision=_PREC)
          + jnp.dot(sdst_ref[:, :_SDIM], w1b_ref[...], precision=_PREC)
          + jnp.dot(aed_ref[...], w1ce_ref[...], precision=_PREC)
          + b1_ref[...])
    h = _silu(h1)
    ms_ref[:, :_SDIM] = (jnp.dot(h, w2s_ref[...], precision=_PREC)
                         + b2s_ref[...]) * C
    g2r = jnp.dot(h, w2g2_ref[...], precision=_PREC) + b2g2_ref[...]
    mv = g2r * rc_ref[...]
    if has_v:
        g1r = jnp.dot(h, w2g1_ref[...], precision=_PREC) + b2g1_ref[...]
        mv = mv + (g1r * gsrc_ref[:, _SDIM:_SDIM + _VF]) * C
    ms_ref[:, _SDIM:_SDIM + _VF] = mv
    ms_ref[:, _SDIM + _VF:] = jnp.full((h.shape[0], _VDIM), pad_val,
                                       jnp.float32)


def _edge_mlp(gsrc, sdst, aed, rc, has_v, wts, pad_val):
    (w1a, w1b, w1ce, b1, w2s, b2s, w2g1, b2g1, w2g2, b2g2) = wts
    full = lambda a: pl.BlockSpec(a.shape, lambda i: tuple(0 for _ in a.shape))
    in_specs = [
        pl.BlockSpec((_BE, 128), lambda i: (i, 0)),
        pl.BlockSpec((_BE, 128), lambda i: (i, 0)),
        pl.BlockSpec((_BE, 48), lambda i: (i, 0)),
        pl.BlockSpec((_BE, 48), lambda i: (i, 0)),
    ]
    args = [gsrc, sdst, aed, rc]
    body = functools.partial(_mlp_body, has_v, pad_val)
    for w in (w1a, w1b, w1ce, b1, w2s, b2s, w2g1, b2g1, w2g2, b2g2):
        in_specs.append(full(w))
        args.append(w)
    return pl.pallas_call(
        body,
        grid=(_E // _BE,),
        in_specs=in_specs,
        out_specs=pl.BlockSpec((_BE, 128), lambda i: (i, 0)),
        out_shape=jax.ShapeDtypeStruct((_E, 128), jnp.float32),
    )(*args)


def _update_body_mlp(tbl_ref, sagg_ref, vagg_ref, cnt_ref,
                     wu1a_ref, wu1b_ref, bu1_ref, wu2_ref, bu2_ref, wv_ref,
                     so_ref, vo_ref):
    cinv = 1.0 / jnp.maximum(cnt_ref[...], 1.0)
    va = vagg_ref[...]
    for j in range(3):
        sl = slice(j * _VDIM, (j + 1) * _VDIM)
        vo_ref[:, sl] = (tbl_ref[:, _SDIM + j * _VDIM:_SDIM + (j + 1) * _VDIM]
                         + jnp.dot(va[:, sl] * cinv, wv_ref[...],
                                   precision=_PREC))
    sn = tbl_ref[:, :_SDIM]
    u = _silu(jnp.dot(sn, wu1a_ref[...], precision=_PREC)
              + jnp.dot(sagg_ref[:, :_SDIM], wu1b_ref[...], precision=_PREC)
              + bu1_ref[...])
    so_ref[...] = sn + jnp.dot(u, wu2_ref[...], precision=_PREC) + bu2_ref[...]


def _update_body_last(tbl_ref, sagg_ref, vagg_ref, cnt_ref, wv_ref,
                      so_ref, vo_ref):
    cinv = 1.0 / jnp.maximum(cnt_ref[...], 1.0)
    va = vagg_ref[...]
    for j in range(3):
        sl = slice(j * _VDIM, (j + 1) * _VDIM)
        vo_ref[:, sl] = (tbl_ref[:, _SDIM + j * _VDIM:_SDIM + (j + 1) * _VDIM]
                         + jnp.dot(va[:, sl] * cinv, wv_ref[...],
                                   precision=_PREC))
    so_ref[...] = tbl_ref[:, :_SDIM] + sagg_ref[:, :_SDIM]


def _node_update(tbl, sagg, vagg, cnt16, wu, wv, last):
    full = lambda a: pl.BlockSpec(a.shape, lambda i: tuple(0 for _ in a.shape))
    in_specs = [
        pl.BlockSpec((_BN, 128), lambda i: (i, 0)),
        pl.BlockSpec((_BN, 128), lambda i: (i, 0)),
        pl.BlockSpec((_BN, 128), lambda i: (i, 0)),
        pl.BlockSpec((_BN, _VDIM), lambda i: (i, 0)),
    ]
    args = [tbl, sagg, vagg, cnt16]
    if last:
        body = _update_body_last
    else:
        body = _update_body_mlp
        for w in wu:
            in_specs.append(full(w))
            args.append(w)
    in_specs.append(full(wv))
    args.append(wv)
    return pl.pallas_call(
        body,
        grid=(_N // _BN,),
        in_specs=in_specs,
        out_specs=[
            pl.BlockSpec((_BN, _SDIM), lambda i: (i, 0)),
            pl.BlockSpec((_BN, _VF), lambda i: (i, 0)),
        ],
        out_shape=[
            jax.ShapeDtypeStruct((_N, _SDIM), jnp.float32),
            jax.ShapeDtypeStruct((_N, _VF), jnp.float32),
        ],
    )(*args)


_WS = 200
_SNW = _E // 16 // _WS
_NT = 3128
_NT_LAST = _N - 15 * _NT
_WG = 200
_GNW = _E // 32 // _WG

_SC_MESH = dict(core_axis_name="c", subcore_axis_name="s")


def _scatter64_body(cb, data_hbm, idx_hbm, z_hbm, out_hbm, acc,
                    ix0, ix1, upd0, upd1, sf0, sf1, ss0, ss1):
    co = lax.axis_index("c")
    sid = lax.axis_index("s")
    colbase = co * 32
    ebase = sid * (_E // 16)

    @pl.when(sid < 15)
    def _():
        pltpu.sync_copy(z_hbm, acc.at[pl.ds(sid * _NT, _NT)])

    @pl.when(sid == 15)
    def _():
        pltpu.sync_copy(z_hbm.at[pl.ds(0, _NT_LAST)],
                        acc.at[pl.ds(15 * _NT, _NT_LAST)])

    plsc.subcore_barrier()

    def fetch(k, ix, upd, sem):
        pltpu.async_copy(idx_hbm.at[pl.ds(ebase + k * _WS, _WS)], ix, sem)
        pltpu.async_copy(
            data_hbm.at[pl.ds(ebase + k * _WS, _WS),
                        pl.ds(cb + colbase, 32)], upd, sem)

    def wait_f(ix, upd, sem):
        pltpu.make_async_copy(idx_hbm.at[pl.ds(0, _WS)], ix, sem).wait()
        pltpu.make_async_copy(
            data_hbm.at[pl.ds(0, _WS), pl.ds(cb + colbase, 32)],
            upd, sem).wait()

    def add(ix, upd, sem):
        pltpu.async_copy(upd, acc.at[ix], sem, add=True)

    def wait_s(ix, upd, sem):
        pltpu.make_async_copy(upd, acc.at[ix], sem).wait()

    fetch(0, ix0, upd0, sf0)
    fetch(1, ix1, upd1, sf1)

    def step(k2, carry):
        a = 2 * k2
        wait_f(ix0, upd0, sf0)
        add(ix0, upd0, ss0)
        wait_f(ix1, upd1, sf1)
        add(ix1, upd1, ss1)
        wait_s(ix0, upd0, ss0)
        fetch(a + 2, ix0, upd0, sf0)
        wait_s(ix1, upd1, ss1)
        fetch(a + 3, ix1, upd1, sf1)
        return carry

    lax.fori_loop(0, _SNW // 2 - 1, step, 0)
    wait_f(ix0, upd0, sf0)
    add(ix0, upd0, ss0)
    wait_f(ix1, upd1, sf1)
    add(ix1, upd1, ss1)
    wait_s(ix0, upd0, ss0)
    wait_s(ix1, upd1, ss1)
    plsc.subcore_barrier()

    @pl.when(sid < 15)
    def _():
        pltpu.sync_copy(acc.at[pl.ds(sid * _NT, _NT)],
                        out_hbm.at[pl.ds(sid * _NT, _NT),
                                   pl.ds(colbase, 32)])

    @pl.when(sid == 15)
    def _():
        pltpu.sync_copy(acc.at[pl.ds(15 * _NT, _NT_LAST)],
                        out_hbm.at[pl.ds(15 * _NT, _NT_LAST),
                                   pl.ds(colbase, 32)])


def _sc_scatter64(data, dst, zeros_t, cb):
    return pl.kernel(
        functools.partial(_scatter64_body, cb),
        out_type=jax.ShapeDtypeStruct((_N, 128), jnp.float32),
        mesh=plsc.VectorSubcoreMesh(**_SC_MESH),
        scratch_types=[
            pltpu.VMEM_SHARED((_N, 32), jnp.float32),
            pltpu.VMEM((_WS,), jnp.int32),
            pltpu.VMEM((_WS,), jnp.int32),
            pltpu.VMEM((_WS, 32), jnp.float32),
            pltpu.VMEM((_WS, 32), jnp.float32),
            pltpu.SemaphoreType.DMA,
            pltpu.SemaphoreType.DMA,
            pltpu.SemaphoreType.DMA,
            pltpu.SemaphoreType.DMA,
        ],
        compiler_params=pltpu.CompilerParams(use_tc_tiling_on_sc=False),
    )(data, dst, zeros_t)


_EPW = _E // 32


def _gather_body(tbl_hbm, idx_hbm, out_hbm, ixf, b0, b1,
                 sg0, sg1, sw0, sw1):
    co = lax.axis_index("c")
    sid = lax.axis_index("s")
    wid = sid * 2 + co
    ebase = wid * _EPW

    pltpu.sync_copy(idx_hbm.at[pl.ds(ebase, _EPW)], ixf)

    def g_fire(k, b, sem):
        pltpu.async_copy(tbl_hbm.at[ixf.at[pl.ds(k * _WG, _WG)]], b, sem)

    def g_wait(b, sem):
        pltpu.make_async_copy(
            tbl_hbm.at[ixf.at[pl.ds(0, _WG)]], b, sem).wait()

    def w_fire(k, b, sem):
        pltpu.async_copy(b, out_hbm.at[pl.ds(ebase + k * _WG, _WG)], sem)

    def w_wait(b, sem):
        pltpu.make_async_copy(b, out_hbm.at[pl.ds(0, _WG)], sem).wait()

    g_fire(0, b0, sg0)
    g_wait(b0, sg0)
    w_fire(0, b0, sw0)
    g_fire(1, b1, sg1)
    w_wait(b0, sw0)
    g_fire(2, b0, sg0)

    def step(k2, carry):
        a = 1 + 2 * k2
        g_wait(b1, sg1)
        w_fire(a, b1, sw1)
        g_wait(b0, sg0)
        w_fire(a + 1, b0, sw0)
        w_wait(b1, sw1)
        g_fire(a + 2, b1, sg1)
        w_wait(b0, sw0)
        g_fire(a + 3, b0, sg0)
        return carry

    lax.fori_loop(0, (_GNW - 1) // 2 - 1, step, 0)
    a = _GNW - 2
    g_wait(b1, sg1)
    w_fire(a, b1, sw1)
    g_wait(b0, sg0)
    w_fire(a + 1, b0, sw0)
    w_wait(b1, sw1)
    w_wait(b0, sw0)


def _sc_gather1(tbl, idx):
    return pl.kernel(
        _gather_body,
        out_type=jax.ShapeDtypeStruct((_E, 128), jnp.float32),
        mesh=plsc.VectorSubcoreMesh(**_SC_MESH),
        scratch_types=[
            pltpu.VMEM((_EPW,), jnp.int32),
            pltpu.VMEM((_WG, 128), jnp.float32),
            pltpu.VMEM((_WG, 128), jnp.float32),
            pltpu.SemaphoreType.DMA,
            pltpu.SemaphoreType.DMA,
            pltpu.SemaphoreType.DMA,
            pltpu.SemaphoreType.DMA,
        ],
    )(tbl, idx)


def kernel(s, v, p, edge_index_local, d_local, a_local, r_local, e_local,
           edge_index_global, d_global, a_global, r_global, e_global, batch,
           ln_gamma, ln_beta, W1, b1, W2, b2, Wu1, bu1, Wu2, bu2, Wv):
    src = edge_index_local[0]
    dst = edge_index_local[1]
    vf = v.reshape(_N, _VF)

    aed, rc = _edge_prep(a_local, e_local, d_local[:, None], r_local)

    zeros_t = jnp.zeros((_NT, 32), jnp.float32)

    cnt16 = None
    for i in range(_L):
        stats = _node_stats(s, vf)
        tbl = _node_apply(s, vf, stats, ln_gamma[i:i + 1], ln_beta[i:i + 1])

        w1 = W1[i]
        w1ce48 = jnp.concatenate(
            [w1[2 * _SDIM:2 * _SDIM + 33], jnp.zeros((15, _HID), jnp.float32)],
            axis=0)
        w2g1r = jnp.tile(W2[i][:, _SDIM:_SDIM + _VDIM], (1, 3))
        b2g1r = jnp.tile(b2[i:i + 1, _SDIM:_SDIM + _VDIM], (1, 3))
        w2g2r = jnp.tile(W2[i][:, _SDIM + _VDIM:], (1, 3))
        b2g2r = jnp.tile(b2[i:i + 1, _SDIM + _VDIM:], (1, 3))
        wts = (w1[:_SDIM], w1[_SDIM:2 * _SDIM], w1ce48, b1[i:i + 1],
               W2[i][:, :_SDIM], b2[i:i + 1, :_SDIM],
               w2g1r, b2g1r, w2g2r, b2g2r)

        gsrc = _sc_gather1(tbl, src)
        sdst = _sc_gather1(tbl, dst)

        msmv = _edge_mlp(gsrc, sdst, aed, rc, i > 0, wts,
                         pad_val=1.0 if i == 0 else 0.0)

        sagg = _sc_scatter64(msmv, dst, zeros_t, cb=0)
        vagg = _sc_scatter64(msmv, dst, zeros_t, cb=64)
        if i == 0:
            cnt16 = vagg[:, _VF:_SDIM]

        wu = (Wu1[i][:_SDIM], Wu1[i][_SDIM:], bu1[i:i + 1], Wu2[i],
              bu2[i:i + 1])
        s, vf = _node_update(tbl, sagg, vagg, cnt16, wu, Wv[i],
                             last=(i == _L - 1))

    return (s, vf.reshape(_N, 3, _VDIM))


# speedup vs baseline: 38.4624x; 1.0301x over previous
import functools

import jax
import jax.numpy as jnp
from jax import lax
from jax.experimental import pallas as pl
from jax.experimental.pallas import tpu as pltpu
from jax.experimental.pallas import tpu_sc as plsc

_N = 50000
_E = 800000
_SDIM = 64
_VDIM = 16
_EDIM = 16
_L = 5
_CUTOFF = 5.0
_HID = 64
_VF = 3 * _VDIM

_BN = 2000
_BE = 2000

_PREC = lax.Precision.DEFAULT


def _stats_body(s_ref, v_ref, o_ref):
    i = pl.program_id(0)
    s = s_ref[...]
    sc = s - jnp.mean(s, axis=1, keepdims=True)
    ps = jnp.sum(sc * sc) * (1.0 / _SDIM)
    vb = v_ref[...]
    pv = jnp.sum(vb * vb) * (1.0 / _VF)

    @pl.when(i == 0)
    def _():
        o_ref[...] = jnp.zeros_like(o_ref)

    col = lax.broadcasted_iota(jnp.int32, (1, 128), 1)
    o_ref[...] += jnp.where(col == 0, ps, 0.0) + jnp.where(col == 1, pv, 0.0)


def _node_stats(s, vf):
    return pl.pallas_call(
        _stats_body,
        grid=(_N // _BN,),
        in_specs=[
            pl.BlockSpec((_BN, _SDIM), lambda i: (i, 0)),
            pl.BlockSpec((_BN, _VF), lambda i: (i, 0)),
        ],
        out_specs=pl.BlockSpec((1, 128), lambda i: (0, 0)),
        out_shape=jax.ShapeDtypeStruct((1, 128), jnp.float32),
    )(s, vf)


def _apply_body(s_ref, v_ref, st_ref, g_ref, b_ref, tbl_ref):
    st = st_ref[...]
    inv_s = lax.rsqrt(st[0, 0] * (1.0 / _N) + 1e-6)
    inv_v = lax.rsqrt(st[0, 1] * (1.0 / _N) + 1e-6)
    s = s_ref[...]
    sc = s - jnp.mean(s, axis=1, keepdims=True)
    tbl_ref[:, :_SDIM] = sc * inv_s * g_ref[...] + b_ref[...]
    tbl_ref[:, _SDIM:_SDIM + _VF] = v_ref[...] * inv_v
    tbl_ref[:, _SDIM + _VF:] = jnp.zeros((s.shape[0], 128 - _SDIM - _VF),
                                         jnp.float32)


def _node_apply(s, vf, stats, gamma, beta):
    return pl.pallas_call(
        _apply_body,
        grid=(_N // _BN,),
        in_specs=[
            pl.BlockSpec((_BN, _SDIM), lambda i: (i, 0)),
            pl.BlockSpec((_BN, _VF), lambda i: (i, 0)),
            pl.BlockSpec((1, 128), lambda i: (0, 0)),
            pl.BlockSpec((1, _SDIM), lambda i: (0, 0)),
            pl.BlockSpec((1, _SDIM), lambda i: (0, 0)),
        ],
        out_specs=pl.BlockSpec((_BN, 128), lambda i: (i, 0)),
        out_shape=jax.ShapeDtypeStruct((_N, 128), jnp.float32),
    )(s, vf, stats, gamma, beta)


def _silu(x):
    return x * jax.nn.sigmoid(x)


def _prep_body(a_ref, e_ref, d_ref, r_ref, aed_ref, rc_ref):
    d = d_ref[...]
    dd = jnp.clip(d, 0.0, _CUTOFF)
    C = 0.5 * (jnp.cos(jnp.pi * dd / _CUTOFF) + 1.0)
    C = jnp.where(d < _CUTOFF, C, 0.0)
    aed_ref[:, 0:_EDIM] = a_ref[...]
    aed_ref[:, _EDIM:2 * _EDIM] = e_ref[...]
    aed_ref[:, 32:33] = d
    aed_ref[:, 33:34] = C
    aed_ref[:, 34:48] = jnp.zeros((d.shape[0], 14), jnp.float32)
    ones16 = jnp.ones((1, _VDIM), jnp.float32)
    for j in range(3):
        rc_ref[:, j * _VDIM:(j + 1) * _VDIM] = (C * r_ref[:, j:j + 1]) * ones16


def _edge_prep(a, e, d2, r):
    return pl.pallas_call(
        _prep_body,
        grid=(_E // _BE,),
        in_specs=[
            pl.BlockSpec((_BE, _EDIM), lambda i: (i, 0)),
            pl.BlockSpec((_BE, _EDIM), lambda i: (i, 0)),
            pl.BlockSpec((_BE, 1), lambda i: (i, 0)),
            pl.BlockSpec((_BE, 3), lambda i: (i, 0)),
        ],
        out_specs=[
            pl.BlockSpec((_BE, 48), lambda i: (i, 0)),
            pl.BlockSpec((_BE, 48), lambda i: (i, 0)),
        ],
        out_shape=[
            jax.ShapeDtypeStruct((_E, 48), jnp.float32),
            jax.ShapeDtypeStruct((_E, 48), jnp.float32),
        ],
    )(a, e, d2, r)


def _mlp_body(has_v, pad_val, gsrc_ref, sdst_ref, aed_ref, rc_ref,
              w1a_ref, w1b_ref, w1ce_ref, b1_ref,
              w2s_ref, b2s_ref, w2g1_ref, b2g1_ref, w2g2_ref, b2g2_ref,
              ms_ref):
    C = aed_ref[:, 33:34]
    h1 = (jnp.dot(gsrc_ref[:, :_SDIM], w1a_ref[...], precision=_PREC)
          + jnp.dot(sdst_ref[:, :_SDIM], w1b_ref[...], precision=_PREC)
          + jnp.dot(aed_ref[...], w1ce_ref[...], precision=_PREC)
          + b1_ref[...])
    h = _silu(h1)
    ms_ref[:, :_SDIM] = (jnp.dot(h, w2s_ref[...], precision=_PREC)
                         + b2s_ref[...]) * C
    g2r = jnp.dot(h, w2g2_ref[...], precision=_PREC) + b2g2_ref[...]
    mv = g2r * rc_ref[...]
    if has_v:
        g1r = jnp.dot(h, w2g1_ref[...], precision=_PREC) + b2g1_ref[...]
        mv = mv + (g1r * gsrc_ref[:, _SDIM:_SDIM + _VF]) * C
    ms_ref[:, _SDIM:_SDIM + _VF] = mv
    ms_ref[:, _SDIM + _VF:] = jnp.full((h.shape[0], _VDIM), pad_val,
                                       jnp.float32)


def _edge_mlp(gsrc, sdst, aed, rc, has_v, wts, pad_val):
    (w1a, w1b, w1ce, b1, w2s, b2s, w2g1, b2g1, w2g2, b2g2) = wts
    full = lambda a: pl.BlockSpec(a.shape, lambda i: tuple(0 for _ in a.shape))
    in_specs = [
        pl.BlockSpec((_BE, 128), lambda i: (i, 0)),
        pl.BlockSpec((_BE, 128), lambda i: (i, 0)),
        pl.BlockSpec((_BE, 48), lambda i: (i, 0)),
        pl.BlockSpec((_BE, 48), lambda i: (i, 0)),
    ]
    args = [gsrc, sdst, aed, rc]
    body = functools.partial(_mlp_body, has_v, pad_val)
    for w in (w1a, w1b, w1ce, b1, w2s, b2s, w2g1, b2g1, w2g2, b2g2):
        in_specs.append(full(w))
        args.append(w)
    return pl.pallas_call(
        body,
        grid=(_E // _BE,),
        in_specs=in_specs,
        out_specs=pl.BlockSpec((_BE, 128), lambda i: (i, 0)),
        out_shape=jax.ShapeDtypeStruct((_E, 128), jnp.float32),
    )(*args)


def _update_body_mlp(tbl_ref, sagg_ref, vagg_ref, cnt_ref,
                     wu1a_ref, wu1b_ref, bu1_ref, wu2_ref, bu2_ref, wv_ref,
                     so_ref, vo_ref):
    cinv = 1.0 / jnp.maximum(cnt_ref[...], 1.0)
    va = vagg_ref[...]
    for j in range(3):
        sl = slice(j * _VDIM, (j + 1) * _VDIM)
        vo_ref[:, sl] = (tbl_ref[:, _SDIM + j * _VDIM:_SDIM + (j + 1) * _VDIM]
                         + jnp.dot(va[:, sl] * cinv, wv_ref[...],
                                   precision=_PREC))
    sn = tbl_ref[:, :_SDIM]
    u = _silu(jnp.dot(sn, wu1a_ref[...], precision=_PREC)
              + jnp.dot(sagg_ref[:, :_SDIM], wu1b_ref[...], precision=_PREC)
              + bu1_ref[...])
    so_ref[...] = sn + jnp.dot(u, wu2_ref[...], precision=_PREC) + bu2_ref[...]


def _update_body_last(tbl_ref, sagg_ref, vagg_ref, cnt_ref, wv_ref,
                      so_ref, vo_ref):
    cinv = 1.0 / jnp.maximum(cnt_ref[...], 1.0)
    va = vagg_ref[...]
    for j in range(3):
        sl = slice(j * _VDIM, (j + 1) * _VDIM)
        vo_ref[:, sl] = (tbl_ref[:, _SDIM + j * _VDIM:_SDIM + (j + 1) * _VDIM]
                         + jnp.dot(va[:, sl] * cinv, wv_ref[...],
                                   precision=_PREC))
    so_ref[...] = tbl_ref[:, :_SDIM] + sagg_ref[:, :_SDIM]


def _node_update(tbl, sagg, vagg, cnt16, wu, wv, last):
    full = lambda a: pl.BlockSpec(a.shape, lambda i: tuple(0 for _ in a.shape))
    in_specs = [
        pl.BlockSpec((_BN, 128), lambda i: (i, 0)),
        pl.BlockSpec((_BN, 128), lambda i: (i, 0)),
        pl.BlockSpec((_BN, 128), lambda i: (i, 0)),
        pl.BlockSpec((_BN, _VDIM), lambda i: (i, 0)),
    ]
    args = [tbl, sagg, vagg, cnt16]
    if last:
        body = _update_body_last
    else:
        body = _update_body_mlp
        for w in wu:
            in_specs.append(full(w))
            args.append(w)
    in_specs.append(full(wv))
    args.append(wv)
    return pl.pallas_call(
        body,
        grid=(_N // _BN,),
        in_specs=in_specs,
        out_specs=[
            pl.BlockSpec((_BN, _SDIM), lambda i: (i, 0)),
            pl.BlockSpec((_BN, _VF), lambda i: (i, 0)),
        ],
        out_shape=[
            jax.ShapeDtypeStruct((_N, _SDIM), jnp.float32),
            jax.ShapeDtypeStruct((_N, _VF), jnp.float32),
        ],
    )(*args)


_WS = 400
_SNW = _E // 16 // _WS
_NT = 3128
_NT_LAST = _N - 15 * _NT
_WG = 200
_GNW = _E // 32 // _WG

_SC_MESH = dict(core_axis_name="c", subcore_axis_name="s")


def _scatter64_body(cb, dw, data_hbm, idx_hbm, z_hbm, out_hbm, acc,
                    ix0, ix1, upd0, upd1, sf0, sf1, ss0, ss1):
    co = lax.axis_index("c")
    sid = lax.axis_index("s")
    colbase = co * dw
    ebase = sid * (_E // 16)

    @pl.when(sid < 15)
    def _():
        pltpu.sync_copy(z_hbm.at[:, pl.ds(0, dw)],
                        acc.at[pl.ds(sid * _NT, _NT)])

    @pl.when(sid == 15)
    def _():
        pltpu.sync_copy(z_hbm.at[pl.ds(0, _NT_LAST), pl.ds(0, dw)],
                        acc.at[pl.ds(15 * _NT, _NT_LAST)])

    plsc.subcore_barrier()

    def fetch(k, ix, upd, sem):
        pltpu.async_copy(idx_hbm.at[pl.ds(ebase + k * _WS, _WS)], ix, sem)
        pltpu.async_copy(
            data_hbm.at[pl.ds(ebase + k * _WS, _WS),
                        pl.ds(cb + colbase, dw)], upd, sem)

    def wait_f(ix, upd, sem):
        pltpu.make_async_copy(idx_hbm.at[pl.ds(0, _WS)], ix, sem).wait()
        pltpu.make_async_copy(
            data_hbm.at[pl.ds(0, _WS), pl.ds(cb + colbase, dw)],
            upd, sem).wait()

    def add(ix, upd, sem):
        pltpu.async_copy(upd, acc.at[ix], sem, add=True)

    def wait_s(ix, upd, sem):
        pltpu.make_async_copy(upd, acc.at[ix], sem).wait()

    fetch(0, ix0, upd0, sf0)
    wait_f(ix0, upd0, sf0)
    add(ix0, upd0, ss0)
    fetch(1, ix1, upd1, sf1)
    wait_s(ix0, upd0, ss0)
    fetch(2, ix0, upd0, sf0)

    def step(k2, carry):
        a = 1 + 2 * k2
        wait_f(ix1, upd1, sf1)
        add(ix1, upd1, ss1)
        wait_f(ix0, upd0, sf0)
        add(ix0, upd0, ss0)
        wait_s(ix1, upd1, ss1)
        fetch(a + 2, ix1, upd1, sf1)
        wait_s(ix0, upd0, ss0)
        fetch(a + 3, ix0, upd0, sf0)
        return carry

    lax.fori_loop(0, (_SNW - 1) // 2 - 1, step, 0)
    wait_f(ix1, upd1, sf1)
    add(ix1, upd1, ss1)
    wait_f(ix0, upd0, sf0)
    add(ix0, upd0, ss0)
    wait_s(ix1, upd1, ss1)
    wait_s(ix0, upd0, ss0)
    plsc.subcore_barrier()

    @pl.when(sid < 15)
    def _():
        pltpu.sync_copy(acc.at[pl.ds(sid * _NT, _NT)],
                        out_hbm.at[pl.ds(sid * _NT, _NT),
                                   pl.ds(colbase, dw)])

    @pl.when(sid == 15)
    def _():
        pltpu.sync_copy(acc.at[pl.ds(15 * _NT, _NT_LAST)],
                        out_hbm.at[pl.ds(15 * _NT, _NT_LAST),
                                   pl.ds(colbase, dw)])


def _sc_scatter64(data, dst, zeros_t, cb, dw=32):
    return pl.kernel(
        functools.partial(_scatter64_body, cb, dw),
        out_type=jax.ShapeDtypeStruct((_N, 128), jnp.float32),
        mesh=plsc.VectorSubcoreMesh(**_SC_MESH),
        scratch_types=[
            pltpu.VMEM_SHARED((_N, dw), jnp.float32),
            pltpu.VMEM((_WS,), jnp.int32),
            pltpu.VMEM((_WS,), jnp.int32),
            pltpu.VMEM((_WS, dw), jnp.float32),
            pltpu.VMEM((_WS, dw), jnp.float32),
            pltpu.SemaphoreType.DMA,
            pltpu.SemaphoreType.DMA,
            pltpu.SemaphoreType.DMA,
            pltpu.SemaphoreType.DMA,
        ],
        compiler_params=pltpu.CompilerParams(use_tc_tiling_on_sc=False),
    )(data, dst, zeros_t)


_EPW = _E // 32


def _gather_body(tbl_hbm, idx_hbm, out_hbm, ixf, b0, b1,
                 sg0, sg1, sw0, sw1):
    co = lax.axis_index("c")
    sid = lax.axis_index("s")
    wid = sid * 2 + co
    ebase = wid * _EPW

    pltpu.sync_copy(idx_hbm.at[pl.ds(ebase, _EPW)], ixf)

    def g_fire(k, b, sem):
        pltpu.async_copy(tbl_hbm.at[ixf.at[pl.ds(k * _WG, _WG)]], b, sem)

    def g_wait(b, sem):
        pltpu.make_async_copy(
            tbl_hbm.at[ixf.at[pl.ds(0, _WG)]], b, sem).wait()

    def w_fire(k, b, sem):
        pltpu.async_copy(b, out_hbm.at[pl.ds(ebase + k * _WG, _WG)], sem)

    def w_wait(b, sem):
        pltpu.make_async_copy(b, out_hbm.at[pl.ds(0, _WG)], sem).wait()

    g_fire(0, b0, sg0)
    g_wait(b0, sg0)
    w_fire(0, b0, sw0)
    g_fire(1, b1, sg1)
    w_wait(b0, sw0)
    g_fire(2, b0, sg0)

    def step(k2, carry):
        a = 1 + 2 * k2
        g_wait(b1, sg1)
        w_fire(a, b1, sw1)
        g_wait(b0, sg0)
        w_fire(a + 1, b0, sw0)
        w_wait(b1, sw1)
        g_fire(a + 2, b1, sg1)
        w_wait(b0, sw0)
        g_fire(a + 3, b0, sg0)
        return carry

    lax.fori_loop(0, (_GNW - 1) // 2 - 1, step, 0)
    a = _GNW - 2
    g_wait(b1, sg1)
    w_fire(a, b1, sw1)
    g_wait(b0, sg0)
    w_fire(a + 1, b0, sw0)
    w_wait(b1, sw1)
    w_wait(b0, sw0)


def _sc_gather1(tbl, idx):
    return pl.kernel(
        _gather_body,
        out_type=jax.ShapeDtypeStruct((_E, 128), jnp.float32),
        mesh=plsc.VectorSubcoreMesh(**_SC_MESH),
        scratch_types=[
            pltpu.VMEM((_EPW,), jnp.int32),
            pltpu.VMEM((_WG, 128), jnp.float32),
            pltpu.VMEM((_WG, 128), jnp.float32),
            pltpu.SemaphoreType.DMA,
            pltpu.SemaphoreType.DMA,
            pltpu.SemaphoreType.DMA,
            pltpu.SemaphoreType.DMA,
        ],
    )(tbl, idx)


def kernel(s, v, p, edge_index_local, d_local, a_local, r_local, e_local,
           edge_index_global, d_global, a_global, r_global, e_global, batch,
           ln_gamma, ln_beta, W1, b1, W2, b2, Wu1, bu1, Wu2, bu2, Wv):
    src = edge_index_local[0]
    dst = edge_index_local[1]
    vf = v.reshape(_N, _VF)

    aed, rc = _edge_prep(a_local, e_local, d_local[:, None], r_local)

    zeros_t = jnp.zeros((_NT, 32), jnp.float32)

    cnt16 = None
    for i in range(_L):
        stats = _node_stats(s, vf)
        tbl = _node_apply(s, vf, stats, ln_gamma[i:i + 1], ln_beta[i:i + 1])

        w1 = W1[i]
        w1ce48 = jnp.concatenate(
            [w1[2 * _SDIM:2 * _SDIM + 33], jnp.zeros((15, _HID), jnp.float32)],
            axis=0)
        w2g1r = jnp.tile(W2[i][:, _SDIM:_SDIM + _VDIM], (1, 3))
        b2g1r = jnp.tile(b2[i:i + 1, _SDIM:_SDIM + _VDIM], (1, 3))
        w2g2r = jnp.tile(W2[i][:, _SDIM + _VDIM:], (1, 3))
        b2g2r = jnp.tile(b2[i:i + 1, _SDIM + _VDIM:], (1, 3))
        wts = (w1[:_SDIM], w1[_SDIM:2 * _SDIM], w1ce48, b1[i:i + 1],
               W2[i][:, :_SDIM], b2[i:i + 1, :_SDIM],
               w2g1r, b2g1r, w2g2r, b2g2r)

        gsrc = _sc_gather1(tbl, src)
        sdst = _sc_gather1(tbl, dst)

        msmv = _edge_mlp(gsrc, sdst, aed, rc, i > 0, wts,
                         pad_val=1.0 if i == 0 else 0.0)

        sagg = _sc_scatter64(msmv, dst, zeros_t, cb=0, dw=32)
        vagg = _sc_scatter64(msmv, dst, zeros_t, cb=64,
                             dw=32 if i == 0 else 24)
        if i == 0:
            cnt16 = vagg[:, _VF:_SDIM]

        wu = (Wu1[i][:_SDIM], Wu1[i][_SDIM:], bu1[i:i + 1], Wu2[i],
              bu2[i:i + 1])
        s, vf = _node_update(tbl, sagg, vagg, cnt16, wu, Wv[i],
                             last=(i == _L - 1))

    return (s, vf.reshape(_N, 3, _VDIM))
